# baseline (speedup 1.0000x reference)
import functools

import jax
import jax.numpy as jnp
from jax import lax
from jax.experimental import pallas as pl
from jax.experimental.pallas import tpu as pltpu
from jax.experimental.pallas import tpu_sc as plsc

_N = 100000
_E = 6400000
_NC = 2
_NS = 16
_NW = _NC * _NS
_LANE = 128
_SUBT = 8
_SUP = _LANE * _SUBT
_HIGH = lax.Precision.HIGHEST
_H3 = lax.Precision.DEFAULT


def _gelu_exact(x):
    return 0.5 * x * (1.0 + lax.erf(x * 0.7071067811865476))


def _make_sc_gather(n_sc):
    mesh = plsc.VectorSubcoreMesh(
        core_axis_name="c", subcore_axis_name="s",
        num_cores=_NC, num_subcores=_NS)

    @functools.partial(
        pl.kernel,
        out_type=(
            jax.ShapeDtypeStruct((n_sc * _SUBT, 8, _LANE), jnp.float32),
            jax.ShapeDtypeStruct((n_sc * _SUBT, 8, _LANE), jnp.float32),
            jax.ShapeDtypeStruct((_NW, 100000), jnp.float32),
        ),
        mesh=mesh,
        scratch_types=[
            pltpu.VMEM((_SUBT, _LANE), jnp.int32),
            pltpu.VMEM((_SUBT, _LANE), jnp.int32),
            pltpu.VMEM((_SUBT // 2, _LANE, 8), jnp.float32),
            pltpu.VMEM((_SUBT // 2, _LANE, 8), jnp.float32),
            pltpu.VMEM((_SUBT, 8, _LANE), jnp.float32),
            pltpu.VMEM((_SUBT, 8, _LANE), jnp.float32),
            pltpu.VMEM((100000,), jnp.float32),
            pltpu.SemaphoreType.DMA,
        ],
        compiler_params=pltpu.CompilerParams(
            use_tc_tiling_on_sc=False, needs_layout_passes=False),
    )
    def gather_kernel(tbl, row3, col3, grow_out, gcol_out, degp_out,
                      idx_r, idx_c, gr, gc, grt, gct, hist, sem):
        wid = lax.axis_index("s") * _NC + lax.axis_index("c")
        nloc = (n_sc - 1 - wid) // _NW + 1
        iota = lax.iota(jnp.int32, 16)
        lanes = [iota + (v * 16) for v in range(_LANE // 16)]
        csp = [jnp.full((16,), c, jnp.int32) for c in range(8)]
        zeros = jnp.zeros((16,), jnp.float32)
        ones = jnp.ones((16,), jnp.float32)

        @pl.loop(0, 100000 // 16)
        def _(i):
            hist[pl.ds(i * 16, 16)] = zeros

        half = _SUBT // 2

        @pl.loop(0, nloc)
        def _(i):
            sc = wid + i * _NW
            pltpu.sync_copy(row3.at[sc], idx_r)
            pltpu.sync_copy(col3.at[sc], idx_c)
            for hh in range(2):
                handles = []
                for j in range(half):
                    jj = hh * half + j
                    handles.append(pltpu.async_copy(
                        tbl.at[idx_r.at[jj]], gr.at[j], sem))
                    handles.append(pltpu.async_copy(
                        tbl.at[idx_c.at[jj]], gc.at[j], sem))
                for j in range(half):
                    jj = hh * half + j
                    for k in range(_LANE // 16):
                        plsc.addupdate_scatter(
                            hist, [idx_r[jj, pl.ds(k * 16, 16)]], ones)
                        plsc.addupdate_scatter(
                            hist, [idx_c[jj, pl.ds(k * 16, 16)]], ones)
                for h in handles:
                    h.wait()
                for j in range(half):
                    jj = hh * half + j
                    for v in range(_LANE // 16):
                        for c in range(8):
                            grt[jj, c, pl.ds(v * 16, 16)] = plsc.load_gather(
                                gr.at[j], [lanes[v], csp[c]])
                            gct[jj, c, pl.ds(v * 16, 16)] = plsc.load_gather(
                                gc.at[j], [lanes[v], csp[c]])
            pltpu.sync_copy(grt, grow_out.at[pl.ds(sc * _SUBT, _SUBT)])
            pltpu.sync_copy(gct, gcol_out.at[pl.ds(sc * _SUBT, _SUBT)])

        pltpu.sync_copy(hist, degp_out.at[wid])

    return gather_kernel


def _make_tc_mlp(n_edges, be):
    nb = n_edges // be
    nt = be // 128

    def unpack(v):
        return jnp.reshape(jnp.transpose(v, (1, 0, 2)), (8, be))

    def body(gr_ref, gc_ref, w1, b1r, w2, b2r, w3, b3r, ef_ref, st_ref):
        gr8 = unpack(gr_ref[...])
        gc8 = unpack(gc_ref[...])
        d = gc8 - gr8
        ndot = jnp.sum(gr8[3:6] * gc8[3:6], axis=0, keepdims=True)
        nsq = jnp.sum(d[0:3] * d[0:3], axis=0, keepdims=True)
        dn = jnp.sqrt(nsq) + 1e-8
        nrd = jnp.sum(gr8[3:6] * d[0:3], axis=0, keepdims=True)
        ncd = jnp.sum(gc8[3:6] * d[0:3], axis=0, keepdims=True)
        eps = 1e-8
        car = jnp.clip(nrd / dn, -1.0 + eps, 1.0 - eps)
        cac = jnp.clip(ncd / dn, -1.0 + eps, 1.0 - eps)
        x8 = jnp.concatenate([d[0:3], ndot, car, cac, d[6:8]], axis=0)
        h = _gelu_exact(jnp.dot(w1[...], x8, precision=_H3) + b1r[...])
        h = _gelu_exact(jnp.dot(w2[...], h, precision=_H3) + b2r[...])
        ef8 = jnp.dot(w3[...], h, precision=_H3) + b3r[...]
        ef_ref[...] = jnp.transpose(jnp.reshape(ef8, (16, nt, 128)), (1, 0, 2))
        s1 = jnp.sum(ef8, axis=1, keepdims=True)
        s2 = jnp.sum(ef8 * ef8, axis=1, keepdims=True)
        st_ref[...] = jnp.concatenate(
            [s1.T[:, None, :], s2.T[:, None, :]], axis=1)

    full = lambda s: pl.BlockSpec(s, lambda i: (0,) * len(s))
    return pl.pallas_call(
        body,
        grid=(nb,),
        in_specs=[
            pl.BlockSpec((nt, 8, 128), lambda i: (i, 0, 0)),
            pl.BlockSpec((nt, 8, 128), lambda i: (i, 0, 0)),
            full((64, 8)), full((64, 1)),
            full((32, 64)), full((32, 1)),
            full((16, 32)), full((16, 1)),
        ],
        out_specs=[
            pl.BlockSpec((nt, 16, 128), lambda i: (i, 0, 0)),
            pl.BlockSpec((1, 2, 16), lambda i: (i, 0, 0)),
        ],
        out_shape=(
            jax.ShapeDtypeStruct((n_edges // 128, 16, 128), jnp.float32),
            jax.ShapeDtypeStruct((nb, 2, 16), jnp.float32),
        ),
    )


def _make_sc_degree(n_sc, n_nodes):
    mesh = plsc.VectorSubcoreMesh(
        core_axis_name="c", subcore_axis_name="s",
        num_cores=_NC, num_subcores=_NS)

    @functools.partial(
        pl.kernel,
        out_type=jax.ShapeDtypeStruct((_NW, n_nodes), jnp.float32),
        mesh=mesh,
        scratch_types=[
            pltpu.VMEM((_SUBT, _LANE), jnp.int32),
            pltpu.VMEM((_SUBT, _LANE), jnp.int32),
            pltpu.VMEM((n_nodes,), jnp.float32),
            pltpu.SemaphoreType.DMA,
        ],
        compiler_params=pltpu.CompilerParams(
            use_tc_tiling_on_sc=False, needs_layout_passes=False),
    )
    def degree_kernel(row3, col3, degp_out, idx_r, idx_c, hist, sem):
        wid = lax.axis_index("s") * _NC + lax.axis_index("c")
        zeros = jnp.zeros((16,), jnp.float32)

        @pl.loop(0, n_nodes // 16)
        def _(i):
            hist[pl.ds(i * 16, 16)] = zeros

        nloc = (n_sc - 1 - wid) // _NW + 1
        ones = jnp.ones((16,), jnp.float32)

        @pl.loop(0, nloc)
        def _(i):
            sc = wid + i * _NW
            pltpu.sync_copy(row3.at[sc], idx_r)
            pltpu.sync_copy(col3.at[sc], idx_c)
            for j in range(_SUBT):
                for k in range(_LANE // 16):
                    plsc.addupdate_scatter(
                        hist, [idx_r[j, pl.ds(k * 16, 16)]], ones)
                    plsc.addupdate_scatter(
                        hist, [idx_c[j, pl.ds(k * 16, 16)]], ones)

        pltpu.sync_copy(hist, degp_out.at[wid])

    return degree_kernel


def _make_sc_scatter(n_sc, n_nodes):
    rpt = n_nodes // _NS
    mesh = plsc.VectorSubcoreMesh(
        core_axis_name="c", subcore_axis_name="s",
        num_cores=_NC, num_subcores=_NS)

    @functools.partial(
        pl.kernel,
        out_type=jax.ShapeDtypeStruct((_NC, n_nodes, 16), jnp.float32),
        mesh=mesh,
        scratch_types=[
            pltpu.VMEM((_SUBT, _LANE), jnp.int32),
            pltpu.VMEM((_SUBT, _LANE), jnp.int32),
            pltpu.VMEM((_SUBT // 2, 16, _LANE), jnp.float32),
            pltpu.VMEM((_SUBT // 2, _LANE, 16), jnp.float32),
            pltpu.VMEM((250, 16), jnp.float32),
            pltpu.VMEM_SHARED((n_nodes, 16), jnp.float32),
            pltpu.SemaphoreType.DMA,
        ],
        compiler_params=pltpu.CompilerParams(
            use_tc_tiling_on_sc=False, needs_layout_passes=False),
    )
    def scatter_kernel(ef3, row3, col3, s2_out,
                       idx_r, idx_c, eft, efb, zbuf, s_sh, sem):
        cid = lax.axis_index("c")
        tid = lax.axis_index("s")
        wid = tid * _NC + cid
        zeros = jnp.zeros((16,), jnp.float32)

        @pl.loop(0, 250)
        def _(i):
            zbuf[i, :] = zeros

        @pl.loop(0, rpt // 250)
        def _(i):
            pltpu.sync_copy(zbuf,
                            s_sh.at[pl.ds(tid * rpt + i * 250, 250)])

        plsc.subcore_barrier()

        nloc = (n_sc - 1 - wid) // _NW + 1
        iota = lax.iota(jnp.int32, 16)
        rowv = [iota + (v * 16) for v in range(_LANE // 16)]
        csp = [jnp.full((16,), c, jnp.int32) for c in range(16)]

        @pl.loop(0, nloc)
        def _(i):
            sc = wid + i * _NW
            pltpu.sync_copy(row3.at[sc], idx_r)
            pltpu.sync_copy(col3.at[sc], idx_c)
            half = _SUBT // 2
            for hh in range(2):
                pltpu.sync_copy(
                    ef3.at[pl.ds(sc * _SUBT + hh * half, half)], eft)
                for j in range(half):
                    for c in range(16):
                        for v in range(_LANE // 16):
                            plsc.store_scatter(
                                efb.at[j], [rowv[v], csp[c]],
                                eft[j, c, pl.ds(v * 16, 16)])
                handles = []
                for j in range(half):
                    blk = efb.at[j]
                    jj = hh * half + j
                    handles.append(pltpu.async_copy(
                        blk, s_sh.at[idx_r.at[jj]], sem, add=True))
                    handles.append(pltpu.async_copy(
                        blk, s_sh.at[idx_c.at[jj]], sem, add=True))
                for h in handles:
                    h.wait()

        plsc.subcore_barrier()
        pltpu.sync_copy(s_sh.at[pl.ds(tid * rpt, rpt)],
                        s2_out.at[cid].at[pl.ds(tid * rpt, rpt)])

    return scatter_kernel


def _make_tc_node_reduce(n_nodes, nb):
    def body(degp, st, deg_ref, sums_ref):
        deg_ref[...] = jnp.sum(degp[...], axis=0, keepdims=True)
        sums_ref[...] = jnp.sum(st[...], axis=0, keepdims=True)

    return pl.pallas_call(
        body,
        out_shape=(
            jax.ShapeDtypeStruct((1, n_nodes), jnp.float32),
            jax.ShapeDtypeStruct((1, 2, 16), jnp.float32),
        ),
    )


def _make_tc_node_apply(n_nodes, n_edges):
    inv_e = 1.0 / float(n_edges)
    rows = n_nodes // 8

    def body(s2, deg8, sums, wp, bpr, ge, be_, gn, bn_, out_ref):
        lane = lax.broadcasted_iota(jnp.int32, (16, 128), 1)
        chan = lax.broadcasted_iota(jnp.int32, (16, 128), 0)
        t16 = jnp.where(lane % 16 == chan, 1.0, 0.0)
        grp = lax.broadcasted_iota(jnp.int32, (8, 128), 1)
        node = lax.broadcasted_iota(jnp.int32, (8, 128), 0)
        r8 = jnp.where(grp // 16 == node, 1.0, 0.0)

        mean = sums[0, 0, :][None, :] * inv_e
        msq = sums[0, 1, :][None, :] * inv_e
        var = msq - mean * mean
        a = ge[...] / jnp.sqrt(var + 1e-5)
        c = be_[...] - mean * a
        a128 = jnp.dot(a, t16, precision=_HIGH)
        c128 = jnp.dot(c, t16, precision=_HIGH)
        deg128 = jnp.dot(deg8[...], r8, precision=_HIGH)

        s = s2[0] + s2[1]
        nf = (s * a128 + deg128 * c128) / jnp.maximum(deg128, 1.0)

        wt = jnp.dot(jnp.dot(t16.T, wp[...], precision=_HIGH), t16,
                     precision=_HIGH)
        blk = lax.broadcasted_iota(jnp.int32, (128, 128), 0) // 16
        blk2 = lax.broadcasted_iota(jnp.int32, (128, 128), 1) // 16
        w128 = jnp.where(blk == blk2, wt, 0.0)
        y = jnp.dot(nf, w128, precision=_HIGH) + jnp.dot(
            bpr[...], t16, precision=_HIGH)

        p128 = jnp.dot(t16.T, t16, precision=_HIGH) * 0.125
        mu = jnp.dot(jnp.mean(y, axis=0, keepdims=True), p128, precision=_HIGH)
        d = y - mu
        varn = jnp.dot(jnp.mean(d * d, axis=0, keepdims=True), p128,
                       precision=_HIGH)
        gn128 = jnp.dot(gn[...], t16, precision=_HIGH)
        bn128 = jnp.dot(bn_[...], t16, precision=_HIGH)
        out_ref[...] = d / jnp.sqrt(varn + 1e-5) * gn128 + bn128

    return pl.pallas_call(
        body,
        out_shape=jax.ShapeDtypeStruct((rows, 128), jnp.float32),
    )


def _run(coords, normals, curvatures, edge_index,
         W1, b1, W2, b2, W3, b3, Wp, bp,
         gamma_e, beta_e, gamma_n, beta_n, n_nodes, n_edges, be=2048):
    n_sc = n_edges // _SUP
    tbl = jnp.concatenate([coords, normals, curvatures[:, :2]], axis=1)
    row3 = edge_index[0].reshape(n_sc, _SUBT, _LANE)
    col3 = edge_index[1].reshape(n_sc, _SUBT, _LANE)

    grow, gcol, degp = _make_sc_gather(n_sc)(tbl, row3, col3)

    ef, stats = _make_tc_mlp(n_edges, be)(
        grow, gcol, W1, b1[:, None], W2, b2[:, None], W3, b3[:, None])

    s2 = _make_sc_scatter(n_sc, n_nodes)(ef, row3, col3)

    deg, sums = _make_tc_node_reduce(n_nodes, n_edges // be)(degp, stats)
    deg8 = deg.reshape(n_nodes // 8, 8)
    s2p = s2.reshape(_NC, n_nodes // 8, 128)
    out = _make_tc_node_apply(n_nodes, n_edges)(
        s2p, deg8, sums, Wp.T, bp[None],
        gamma_e[None], beta_e[None], gamma_n[None], beta_n[None])
    return out.reshape(n_nodes, 16)


def kernel(coords, normals, curvatures, edge_index,
           W1, b1, W2, b2, W3, b3, Wp, bp,
           gamma_e, beta_e, gamma_n, beta_n):
    return _run(coords, normals, curvatures, edge_index,
                W1, b1, W2, b2, W3, b3, Wp, bp,
                gamma_e, beta_e, gamma_n, beta_n, _N, _E, be=12800)


# speedup vs baseline: 1.0341x; 1.0341x over previous
import functools

import jax
import jax.numpy as jnp
from jax import lax
from jax.experimental import pallas as pl
from jax.experimental.pallas import tpu as pltpu
from jax.experimental.pallas import tpu_sc as plsc

_N = 100000
_E = 6400000
_NC = 2
_NS = 16
_NW = _NC * _NS
_LANE = 128
_SUBT = 8
_SUP = _LANE * _SUBT
_HIGH = lax.Precision.HIGHEST
_MLP = lax.Precision.DEFAULT


def _gelu_exact(x):
    return 0.5 * x * (1.0 + lax.erf(x * 0.7071067811865476))


def _make_sc_gather(n_sc):
    mesh = plsc.VectorSubcoreMesh(
        core_axis_name="c", subcore_axis_name="s",
        num_cores=_NC, num_subcores=_NS)

    @functools.partial(
        pl.kernel,
        out_type=(
            jax.ShapeDtypeStruct((n_sc * _SUBT, 8, _LANE), jnp.float32),
            jax.ShapeDtypeStruct((n_sc * _SUBT, 8, _LANE), jnp.float32),
        ),
        mesh=mesh,
        scratch_types=[
            pltpu.VMEM((_SUBT, _LANE), jnp.int32),
            pltpu.VMEM((_SUBT, _LANE), jnp.int32),
            pltpu.VMEM((_SUBT, _LANE, 8), jnp.float32),
            pltpu.VMEM((_SUBT, _LANE, 8), jnp.float32),
            pltpu.VMEM((_SUBT, 8, _LANE), jnp.float32),
            pltpu.VMEM((_SUBT, 8, _LANE), jnp.float32),
            pltpu.SemaphoreType.DMA,
        ],
        compiler_params=pltpu.CompilerParams(
            use_tc_tiling_on_sc=False, needs_layout_passes=False),
    )
    def gather_kernel(tbl, row3, col3, grow_out, gcol_out,
                      idx_r, idx_c, gr, gc, grt, gct, sem):
        wid = lax.axis_index("s") * _NC + lax.axis_index("c")
        nloc = (n_sc - 1 - wid) // _NW + 1
        iota = lax.iota(jnp.int32, 16)
        lanes = [iota + (v * 16) for v in range(_LANE // 16)]
        csp = [jnp.full((16,), c, jnp.int32) for c in range(8)]

        @pl.loop(0, nloc)
        def _(i):
            sc = wid + i * _NW
            pltpu.sync_copy(row3.at[sc], idx_r)
            pltpu.sync_copy(col3.at[sc], idx_c)
            handles = []
            for j in range(_SUBT):
                handles.append(pltpu.async_copy(
                    tbl.at[idx_r.at[j]], gr.at[j], sem))
                handles.append(pltpu.async_copy(
                    tbl.at[idx_c.at[j]], gc.at[j], sem))
            for h in handles:
                h.wait()
            for j in range(_SUBT):
                for v in range(_LANE // 16):
                    for c in range(8):
                        grt[j, c, pl.ds(v * 16, 16)] = plsc.load_gather(
                            gr.at[j], [lanes[v], csp[c]])
                        gct[j, c, pl.ds(v * 16, 16)] = plsc.load_gather(
                            gc.at[j], [lanes[v], csp[c]])
            pltpu.sync_copy(grt, grow_out.at[pl.ds(sc * _SUBT, _SUBT)])
            pltpu.sync_copy(gct, gcol_out.at[pl.ds(sc * _SUBT, _SUBT)])

    return gather_kernel


def _make_tc_mlp(n_edges, be):
    nb = n_edges // be
    nt = be // 128

    def unpack(v):
        return jnp.reshape(jnp.transpose(v, (1, 0, 2)), (8, be))

    def body(gr_ref, gc_ref, w1, b1r, w2, b2r, w3, b3r, ef_ref, st_ref):
        gr8 = unpack(gr_ref[...])
        gc8 = unpack(gc_ref[...])
        d = gc8 - gr8
        ndot = jnp.sum(gr8[3:6] * gc8[3:6], axis=0, keepdims=True)
        nsq = jnp.sum(d[0:3] * d[0:3], axis=0, keepdims=True)
        dn = jnp.sqrt(nsq) + 1e-8
        nrd = jnp.sum(gr8[3:6] * d[0:3], axis=0, keepdims=True)
        ncd = jnp.sum(gc8[3:6] * d[0:3], axis=0, keepdims=True)
        eps = 1e-8
        car = jnp.clip(nrd / dn, -1.0 + eps, 1.0 - eps)
        cac = jnp.clip(ncd / dn, -1.0 + eps, 1.0 - eps)
        x8 = jnp.concatenate([d[0:3], ndot, car, cac, d[6:8]], axis=0)
        h = _gelu_exact(jnp.dot(w1[...], x8, precision=_MLP) + b1r[...])
        h = _gelu_exact(jnp.dot(w2[...], h, precision=_MLP) + b2r[...])
        ef8 = jnp.dot(w3[...], h, precision=_MLP) + b3r[...]
        ef_ref[...] = jnp.transpose(jnp.reshape(ef8, (16, nt, 128)), (1, 0, 2))
        s1 = jnp.sum(ef8, axis=1, keepdims=True)
        s2 = jnp.sum(ef8 * ef8, axis=1, keepdims=True)
        st_ref[...] = jnp.concatenate(
            [s1.T[:, None, :], s2.T[:, None, :]], axis=1)

    full = lambda s: pl.BlockSpec(s, lambda i: (0,) * len(s))
    return pl.pallas_call(
        body,
        grid=(nb,),
        in_specs=[
            pl.BlockSpec((nt, 8, 128), lambda i: (i, 0, 0)),
            pl.BlockSpec((nt, 8, 128), lambda i: (i, 0, 0)),
            full((64, 8)), full((64, 1)),
            full((32, 64)), full((32, 1)),
            full((16, 32)), full((16, 1)),
        ],
        out_specs=[
            pl.BlockSpec((nt, 16, 128), lambda i: (i, 0, 0)),
            pl.BlockSpec((1, 2, 16), lambda i: (i, 0, 0)),
        ],
        out_shape=(
            jax.ShapeDtypeStruct((n_edges // 128, 16, 128), jnp.float32),
            jax.ShapeDtypeStruct((nb, 2, 16), jnp.float32),
        ),
    )


def _make_sc_degree(n_sc, n_nodes):
    mesh = plsc.VectorSubcoreMesh(
        core_axis_name="c", subcore_axis_name="s",
        num_cores=_NC, num_subcores=_NS)

    @functools.partial(
        pl.kernel,
        out_type=jax.ShapeDtypeStruct((_NW, n_nodes), jnp.float32),
        mesh=mesh,
        scratch_types=[
            pltpu.VMEM((_SUBT, _LANE), jnp.int32),
            pltpu.VMEM((_SUBT, _LANE), jnp.int32),
            pltpu.VMEM((n_nodes,), jnp.float32),
            pltpu.SemaphoreType.DMA,
        ],
        compiler_params=pltpu.CompilerParams(
            use_tc_tiling_on_sc=False, needs_layout_passes=False),
    )
    def degree_kernel(row3, col3, degp_out, idx_r, idx_c, hist, sem):
        wid = lax.axis_index("s") * _NC + lax.axis_index("c")
        zeros = jnp.zeros((16,), jnp.float32)

        @pl.loop(0, n_nodes // 16)
        def _(i):
            hist[pl.ds(i * 16, 16)] = zeros

        nloc = (n_sc - 1 - wid) // _NW + 1
        ones = jnp.ones((16,), jnp.float32)

        @pl.loop(0, nloc)
        def _(i):
            sc = wid + i * _NW
            pltpu.sync_copy(row3.at[sc], idx_r)
            pltpu.sync_copy(col3.at[sc], idx_c)
            for j in range(_SUBT):
                for k in range(_LANE // 16):
                    plsc.addupdate_scatter(
                        hist, [idx_r[j, pl.ds(k * 16, 16)]], ones)
                    plsc.addupdate_scatter(
                        hist, [idx_c[j, pl.ds(k * 16, 16)]], ones)

        pltpu.sync_copy(hist, degp_out.at[wid])

    return degree_kernel


def _make_sc_scatter(n_sc, n_nodes):
    rpt = n_nodes // _NS
    mesh = plsc.VectorSubcoreMesh(
        core_axis_name="c", subcore_axis_name="s",
        num_cores=_NC, num_subcores=_NS)

    @functools.partial(
        pl.kernel,
        out_type=jax.ShapeDtypeStruct((_NC, n_nodes, 16), jnp.float32),
        mesh=mesh,
        scratch_types=[
            pltpu.VMEM((_SUBT, _LANE), jnp.int32),
            pltpu.VMEM((_SUBT, _LANE), jnp.int32),
            pltpu.VMEM((_SUBT // 2, 16, _LANE), jnp.float32),
            pltpu.VMEM((_SUBT // 2, _LANE, 16), jnp.float32),
            pltpu.VMEM((250, 16), jnp.float32),
            pltpu.VMEM_SHARED((n_nodes, 16), jnp.float32),
            pltpu.SemaphoreType.DMA,
        ],
        compiler_params=pltpu.CompilerParams(
            use_tc_tiling_on_sc=False, needs_layout_passes=False),
    )
    def scatter_kernel(ef3, row3, col3, s2_out,
                       idx_r, idx_c, eft, efb, zbuf, s_sh, sem):
        cid = lax.axis_index("c")
        tid = lax.axis_index("s")
        wid = tid * _NC + cid
        zeros = jnp.zeros((16,), jnp.float32)

        @pl.loop(0, 250)
        def _(i):
            zbuf[i, :] = zeros

        @pl.loop(0, rpt // 250)
        def _(i):
            pltpu.sync_copy(zbuf,
                            s_sh.at[pl.ds(tid * rpt + i * 250, 250)])

        plsc.subcore_barrier()

        nloc = (n_sc - 1 - wid) // _NW + 1
        iota = lax.iota(jnp.int32, 16)
        rowv = [iota + (v * 16) for v in range(_LANE // 16)]
        csp = [jnp.full((16,), c, jnp.int32) for c in range(16)]

        @pl.loop(0, nloc)
        def _(i):
            sc = wid + i * _NW
            pltpu.sync_copy(row3.at[sc], idx_r)
            pltpu.sync_copy(col3.at[sc], idx_c)
            half = _SUBT // 2
            for hh in range(2):
                pltpu.sync_copy(
                    ef3.at[pl.ds(sc * _SUBT + hh * half, half)], eft)
                for j in range(half):
                    for c in range(16):
                        for v in range(_LANE // 16):
                            plsc.store_scatter(
                                efb.at[j], [rowv[v], csp[c]],
                                eft[j, c, pl.ds(v * 16, 16)])
                handles = []
                for j in range(half):
                    blk = efb.at[j]
                    jj = hh * half + j
                    handles.append(pltpu.async_copy(
                        blk, s_sh.at[idx_r.at[jj]], sem, add=True))
                    handles.append(pltpu.async_copy(
                        blk, s_sh.at[idx_c.at[jj]], sem, add=True))
                for h in handles:
                    h.wait()

        plsc.subcore_barrier()
        pltpu.sync_copy(s_sh.at[pl.ds(tid * rpt, rpt)],
                        s2_out.at[cid].at[pl.ds(tid * rpt, rpt)])

    return scatter_kernel


def _make_tc_node_reduce(n_nodes, nb):
    def body(degp, st, deg_ref, sums_ref):
        deg_ref[...] = jnp.sum(degp[...], axis=0, keepdims=True)
        sums_ref[...] = jnp.sum(st[...], axis=0, keepdims=True)

    return pl.pallas_call(
        body,
        out_shape=(
            jax.ShapeDtypeStruct((1, n_nodes), jnp.float32),
            jax.ShapeDtypeStruct((1, 2, 16), jnp.float32),
        ),
    )


def _make_tc_node_apply(n_nodes, n_edges):
    inv_e = 1.0 / float(n_edges)
    rows = n_nodes // 8

    def body(s2, deg8, sums, wp, bpr, ge, be_, gn, bn_, out_ref):
        lane = lax.broadcasted_iota(jnp.int32, (16, 128), 1)
        chan = lax.broadcasted_iota(jnp.int32, (16, 128), 0)
        t16 = jnp.where(lane % 16 == chan, 1.0, 0.0)
        grp = lax.broadcasted_iota(jnp.int32, (8, 128), 1)
        node = lax.broadcasted_iota(jnp.int32, (8, 128), 0)
        r8 = jnp.where(grp // 16 == node, 1.0, 0.0)

        mean = sums[0, 0, :][None, :] * inv_e
        msq = sums[0, 1, :][None, :] * inv_e
        var = msq - mean * mean
        a = ge[...] / jnp.sqrt(var + 1e-5)
        c = be_[...] - mean * a
        a128 = jnp.dot(a, t16, precision=_HIGH)
        c128 = jnp.dot(c, t16, precision=_HIGH)
        deg128 = jnp.dot(deg8[...], r8, precision=_HIGH)

        s = s2[0] + s2[1]
        nf = (s * a128 + deg128 * c128) / jnp.maximum(deg128, 1.0)

        wt = jnp.dot(jnp.dot(t16.T, wp[...], precision=_HIGH), t16,
                     precision=_HIGH)
        blk = lax.broadcasted_iota(jnp.int32, (128, 128), 0) // 16
        blk2 = lax.broadcasted_iota(jnp.int32, (128, 128), 1) // 16
        w128 = jnp.where(blk == blk2, wt, 0.0)
        y = jnp.dot(nf, w128, precision=_HIGH) + jnp.dot(
            bpr[...], t16, precision=_HIGH)

        p128 = jnp.dot(t16.T, t16, precision=_HIGH) * 0.125
        mu = jnp.dot(jnp.mean(y, axis=0, keepdims=True), p128, precision=_HIGH)
        d = y - mu
        varn = jnp.dot(jnp.mean(d * d, axis=0, keepdims=True), p128,
                       precision=_HIGH)
        gn128 = jnp.dot(gn[...], t16, precision=_HIGH)
        bn128 = jnp.dot(bn_[...], t16, precision=_HIGH)
        out_ref[...] = d / jnp.sqrt(varn + 1e-5) * gn128 + bn128

    return pl.pallas_call(
        body,
        out_shape=jax.ShapeDtypeStruct((rows, 128), jnp.float32),
    )


def _run(coords, normals, curvatures, edge_index,
         W1, b1, W2, b2, W3, b3, Wp, bp,
         gamma_e, beta_e, gamma_n, beta_n, n_nodes, n_edges, be=2048):
    n_sc = n_edges // _SUP
    tbl = jnp.concatenate([coords, normals, curvatures[:, :2]], axis=1)
    row3 = edge_index[0].reshape(n_sc, _SUBT, _LANE)
    col3 = edge_index[1].reshape(n_sc, _SUBT, _LANE)

    grow, gcol = _make_sc_gather(n_sc)(tbl, row3, col3)

    ef, stats = _make_tc_mlp(n_edges, be)(
        grow, gcol, W1, b1[:, None], W2, b2[:, None], W3, b3[:, None])

    degp = _make_sc_degree(n_sc, n_nodes)(row3, col3)
    s2 = _make_sc_scatter(n_sc, n_nodes)(ef, row3, col3)

    deg, sums = _make_tc_node_reduce(n_nodes, n_edges // be)(degp, stats)
    deg8 = deg.reshape(n_nodes // 8, 8)
    s2p = s2.reshape(_NC, n_nodes // 8, 128)
    out = _make_tc_node_apply(n_nodes, n_edges)(
        s2p, deg8, sums, Wp.T, bp[None],
        gamma_e[None], beta_e[None], gamma_n[None], beta_n[None])
    return out.reshape(n_nodes, 16)


def kernel(coords, normals, curvatures, edge_index,
           W1, b1, W2, b2, W3, b3, Wp, bp,
           gamma_e, beta_e, gamma_n, beta_n):
    return _run(coords, normals, curvatures, edge_index,
                W1, b1, W2, b2, W3, b3, Wp, bp,
                gamma_e, beta_e, gamma_n, beta_n, _N, _E, be=12800)
